# baseline (speedup 1.0000x reference)
import functools

import jax
import jax.numpy as jnp
from jax import lax
from jax.experimental import pallas as pl
from jax.experimental.pallas import tpu as pltpu
from jax.experimental.pallas import tpu_sc as plsc

N = 10000
E = 320000
D = 128
DE = 16
DU = 128
H = 128
NG = 16

E1 = 166400
E2 = E - E1
CH = 80
GCH = 40
NP = 10240
NHI = NP // 128
EB = 1600
NB = 2000

ZR = 128
HALF = NP // 2
ACC = 6144
TRASH = HALF

f32 = jnp.float32
bf16 = jnp.bfloat16


def _pack_pair(a):
    u = lax.bitcast_convert_type(a.astype(bf16), jnp.uint16).astype(jnp.uint32)
    w = a.shape[1]
    return u[:, : w // 2] | (u[:, w // 2:] << 16)


def _unpack_lo(p):
    return lax.bitcast_convert_type(p << 16, f32)


def _unpack_hi(p):
    return lax.bitcast_convert_type(p & jnp.uint32(0xFFFF0000), f32)


def _ln(h, s, b):
    m = jnp.mean(h, axis=-1, keepdims=True)
    v = jnp.mean((h - m) ** 2, axis=-1, keepdims=True)
    return (h - m) * lax.rsqrt(v + 1e-5) * s + b


def _pre_body(x_ref, b16_ref, u_ref, wsrc_ref, wdst_ref, wu_ref, b0e_ref,
              wn1d_ref, b0n1_ref, wn2u_ref, b0n2_ref,
              prow_ref, pcol_ref, pu2_ref):
    oh = (b16_ref[...] == lax.broadcasted_iota(jnp.int32, (NB, NG), 1)).astype(f32)
    u = u_ref[...]
    x = x_ref[...]
    uproj_e = jnp.dot(u, wu_ref[...], preferred_element_type=f32)
    prow = (jnp.dot(x, wsrc_ref[...], preferred_element_type=f32)
            + jnp.dot(oh, uproj_e, preferred_element_type=f32)
            + b0e_ref[...])
    col_a = jnp.dot(x, wdst_ref[...], preferred_element_type=f32)
    col_b = jnp.dot(x, wn1d_ref[...], preferred_element_type=f32) + b0n1_ref[...]
    prow_ref[...] = prow
    pcol_ref[...] = _pack_pair(jnp.concatenate([col_a, col_b], axis=1))
    pu2_ref[...] = (jnp.dot(oh, jnp.dot(u, wn2u_ref[...], preferred_element_type=f32),
                            preferred_element_type=f32) + b0n2_ref[...])


def _pre_call(x, b16, u, wsrc, wdst, wu, b0e, wn1d, b0n1, wn2u, b0n2):
    full = lambda shape: pl.BlockSpec(shape, lambda i: (0,) * len(shape))
    return pl.pallas_call(
        _pre_body,
        grid=(N // NB,),
        in_specs=[
            pl.BlockSpec((NB, D), lambda i: (i, 0)),
            pl.BlockSpec((NB, NG), lambda i: (i, 0)),
            full((NG, DU)), full((D, H)), full((D, H)), full((DU, H)),
            full((1, H)), full((D, H)), full((1, H)), full((DU, H)), full((1, H)),
        ],
        out_specs=[
            pl.BlockSpec((NB, H), lambda i: (i, 0)),
            pl.BlockSpec((NB, H), lambda i: (i, 0)),
            pl.BlockSpec((NB, H), lambda i: (i, 0)),
        ],
        out_shape=[
            jax.ShapeDtypeStruct((N, H), f32),
            jax.ShapeDtypeStruct((N, H), jnp.uint32),
            jax.ShapeDtypeStruct((N, H), f32),
        ],
    )(x, b16, u, wsrc, wdst, wu, b0e, wn1d, b0n1, wn2u, b0n2)


def _sc_gather(pre_row, pre_col, row, col, ne):
    perw = ne // 32
    nch = perw // GCH
    npair = nch // 2 - 1
    mesh = plsc.VectorSubcoreMesh(core_axis_name="c", subcore_axis_name="s")

    @functools.partial(
        pl.kernel,
        out_type=(jax.ShapeDtypeStruct((ne, H), f32),
                  jax.ShapeDtypeStruct((ne, H), jnp.uint32)),
        mesh=mesh,
        scratch_types=[
            pltpu.VMEM((perw,), jnp.int32),
            pltpu.VMEM((perw,), jnp.int32),
            pltpu.VMEM((GCH, H), f32),
            pltpu.VMEM((GCH, H), f32),
            pltpu.VMEM((GCH, H), jnp.uint32),
            pltpu.VMEM((GCH, H), jnp.uint32),
        ] + [pltpu.SemaphoreType.DMA] * 8,
    )
    def gk(prer_hbm, prec_hbm, row_hbm, col_hbm, outr_hbm, outc_hbm,
           ridx, cidx, rb0, rb1, cb0, cb1, sr0, sr1, sc0, sc1, wr0, wr1, wc0, wc1):
        wid = lax.axis_index("s") * 2 + lax.axis_index("c")
        base = wid * perw
        pltpu.sync_copy(row_hbm.at[pl.ds(base, perw)], ridx)
        pltpu.sync_copy(col_hbm.at[pl.ds(base, perw)], cidx)
        rbufs, cbufs = (rb0, rb1), (cb0, cb1)
        srs, scs = (sr0, sr1), (sc0, sc1)
        wrs, wcs = (wr0, wr1), (wc0, wc1)

        def issue_g(j, s):
            pltpu.async_copy(prer_hbm.at[ridx.at[pl.ds(j * GCH, GCH)]], rbufs[s], srs[s])
            pltpu.async_copy(prec_hbm.at[cidx.at[pl.ds(j * GCH, GCH)]], cbufs[s], scs[s])

        def wait_g(s):
            pltpu.make_async_copy(prer_hbm.at[ridx.at[pl.ds(0, GCH)]], rbufs[s], srs[s]).wait()
            pltpu.make_async_copy(prec_hbm.at[cidx.at[pl.ds(0, GCH)]], cbufs[s], scs[s]).wait()

        def start_wb(j, s):
            pltpu.async_copy(rbufs[s], outr_hbm.at[pl.ds(base + j * GCH, GCH)], wrs[s])
            pltpu.async_copy(cbufs[s], outc_hbm.at[pl.ds(base + j * GCH, GCH)], wcs[s])

        def wait_wb(s):
            pltpu.make_async_copy(rbufs[s], outr_hbm.at[pl.ds(base, GCH)], wrs[s]).wait()
            pltpu.make_async_copy(cbufs[s], outc_hbm.at[pl.ds(base, GCH)], wcs[s]).wait()

        issue_g(0, 0)
        issue_g(1, 1)

        def body(jj, carry):
            j0 = jj * 2
            wait_g(0)
            start_wb(j0, 0)
            wait_g(1)
            start_wb(j0 + 1, 1)
            wait_wb(0)
            issue_g(j0 + 2, 0)
            wait_wb(1)
            issue_g(j0 + 3, 1)
            return carry

        lax.fori_loop(0, npair, body, 0)
        wait_g(0)
        start_wb(nch - 2, 0)
        wait_g(1)
        start_wb(nch - 1, 1)
        wait_wb(0)
        wait_wb(1)

    return gk(pre_row, pre_col, row, col)


def _edge_body(gr_ref, gc_ref, ea_ref, row_ref, w0ea_ref, w1e_ref, b1e_ref,
               se_ref, be_ref, w0n1e_ref, w1n1_ref, b1n1_ref, sn1_ref, bn1_ref,
               enew_ref, nh_ref, cnt_ref):
    gcp = gc_ref[...]
    gca = _unpack_lo(gcp)
    gcb = _unpack_hi(gcp)
    eap = jnp.dot(ea_ref[...], w0ea_ref[...], preferred_element_type=f32)
    h0 = jax.nn.gelu(gr_ref[...] + gca + eap)
    h1 = jax.nn.gelu(jnp.dot(h0, w1e_ref[...], preferred_element_type=f32) + b1e_ref[...])
    e_new = _ln(h1 + h0, se_ref[...], be_ref[...])
    m0 = jax.nn.gelu(gcb + jnp.dot(e_new, w0n1e_ref[...], preferred_element_type=f32))
    m1 = jax.nn.gelu(jnp.dot(m0, w1n1_ref[...], preferred_element_type=f32) + b1n1_ref[...])
    enew_ref[...] = e_new
    nh_ref[...] = _ln(m1 + m0, sn1_ref[...], bn1_ref[...])
    i = pl.program_id(0)

    @pl.when(i == 0)
    def _():
        cnt_ref[...] = jnp.zeros((NHI, 128), f32)

    r2 = row_ref[...]
    oh_hi = ((r2 >> 7) == lax.broadcasted_iota(jnp.int32, (EB, NHI), 1)).astype(f32)
    oh_lo = ((r2 & 127) == lax.broadcasted_iota(jnp.int32, (EB, 128), 1)).astype(f32)
    cnt_ref[...] += lax.dot_general(oh_hi, oh_lo, (((0,), (0,)), ((), ())),
                                    preferred_element_type=f32)


def _edge_call(gr, gc, edge_attr, row2d, w0ea, w1e, b1e, se, be,
               w0n1e, w1n1, b1n1, sn1, bn1, ne):
    full = lambda shape: pl.BlockSpec(shape, lambda i: (0,) * len(shape))
    return pl.pallas_call(
        _edge_body,
        grid=(ne // EB,),
        in_specs=[
            pl.BlockSpec((EB, H), lambda i: (i, 0)),
            pl.BlockSpec((EB, H), lambda i: (i, 0)),
            pl.BlockSpec((EB, DE), lambda i: (i, 0)),
            pl.BlockSpec((EB, 1), lambda i: (i, 0)),
            full((DE, H)), full((H, H)), full((1, H)), full((1, H)), full((1, H)),
            full((H, H)), full((H, H)), full((1, H)), full((1, H)), full((1, H)),
        ],
        out_specs=[
            pl.BlockSpec((EB, H), lambda i: (i, 0)),
            pl.BlockSpec((EB, H), lambda i: (i, 0)),
            pl.BlockSpec((NHI, 128), lambda i: (0, 0)),
        ],
        out_shape=[
            jax.ShapeDtypeStruct((ne, H), f32),
            jax.ShapeDtypeStruct((ne, H), f32),
            jax.ShapeDtypeStruct((NHI, 128), f32),
        ],
    )(gr, gc, edge_attr, row2d, w0ea, w1e, b1e, se, be, w0n1e, w1n1, b1n1, sn1, bn1)


def _sc_sums(nh, row, zrow, ne):
    pert = ne // 16
    nchs = pert // CH
    mesh = plsc.VectorSubcoreMesh(core_axis_name="c", subcore_axis_name="s")

    @functools.partial(
        pl.kernel,
        out_type=jax.ShapeDtypeStruct((NP, H), f32),
        mesh=mesh,
        scratch_types=[
            pltpu.VMEM((pert,), jnp.int32),
            pltpu.VMEM((CH,), jnp.int32),
            pltpu.VMEM((CH, H), f32),
            pltpu.VMEM((CH, H), f32),
            pltpu.VMEM((ZR, H), f32),
            pltpu.VMEM_SHARED((ACC, H), f32),
            pltpu.SemaphoreType.DMA,
            pltpu.SemaphoreType.DMA,
        ],
    )
    def sk(nh_hbm, row_hbm, zrow_hbm, outs_hbm,
           idxall, idxb2, vb0, vb1, zrb, acc_sh, sl0, sl1):
        cid = lax.axis_index("c")
        sid = lax.axis_index("s")
        base = sid * pert
        lo = cid * HALF
        pltpu.sync_copy(zrow_hbm, zrb)
        pltpu.sync_copy(row_hbm.at[pl.ds(base, pert)], idxall)

        def zbody(k, carry):
            pltpu.sync_copy(zrb, acc_sh.at[pl.ds(sid * (ACC // 16) + k * ZR, ZR)])
            return carry

        lax.fori_loop(0, ACC // 16 // ZR, zbody, 0)
        plsc.subcore_barrier()
        vbufs, sls = (vb0, vb1), (sl0, sl1)

        def load(j, s):
            pltpu.async_copy(nh_hbm.at[pl.ds(base + j * CH, CH)], vbufs[s], sls[s])

        def wait_load(s):
            pltpu.make_async_copy(nh_hbm.at[pl.ds(base, CH)], vbufs[s], sls[s]).wait()

        def remap(j):
            for k in range(CH // 16):
                v = idxall[pl.ds(j * CH + k * 16, 16)] - lo
                oob = (v < 0) | (v >= HALF)
                idxb2[pl.ds(k * 16, 16)] = jnp.where(oob, TRASH, v)

        def scat(s):
            pltpu.sync_copy(vbufs[s], acc_sh.at[idxb2], add=True)

        load(0, 0)

        def body(jj, carry):
            j0 = jj * 2
            wait_load(0)
            load(j0 + 1, 1)
            remap(j0)
            scat(0)
            wait_load(1)
            load(j0 + 2, 0)
            remap(j0 + 1)
            scat(1)
            return carry

        lax.fori_loop(0, nchs // 2 - 1, body, 0)
        wait_load(0)
        load(nchs - 1, 1)
        remap(nchs - 2)
        scat(0)
        wait_load(1)
        remap(nchs - 1)
        scat(1)
        plsc.subcore_barrier()

        def dbody(k, carry):
            src = sid * (HALF // 16) + k * CH
            pltpu.sync_copy(acc_sh.at[pl.ds(src, CH)], vb0)
            pltpu.sync_copy(vb0, outs_hbm.at[pl.ds(cid * HALF + src, CH)])
            return carry

        lax.fori_loop(0, HALF // 16 // CH, dbody, 0)

    return sk(nh, row, zrow)


def _post_body(sa_ref, sb_ref, cnt_ref, pu2_ref, b16_ref, u_ref,
               w0n2a_ref, w1n2_ref, b1n2_ref, sn2_ref, bn2_ref,
               w0gu_ref, w0gm_ref, b0g_ref, w1g_ref, b1g_ref, sg_ref, bg_ref,
               xnew_ref, unew_ref):
    s = sa_ref[:N, :] + sb_ref[:N, :]
    c = cnt_ref[:N, :]
    agg = s * (1.0 / jnp.maximum(c, 1.0))
    a0 = jax.nn.gelu(jnp.dot(agg, w0n2a_ref[...], preferred_element_type=f32) + pu2_ref[...])
    a1 = jax.nn.gelu(jnp.dot(a0, w1n2_ref[...], preferred_element_type=f32) + b1n2_ref[...])
    x_new = _ln(a1 + a0, sn2_ref[...], bn2_ref[...])
    xnew_ref[...] = x_new
    oh = (b16_ref[...] == lax.broadcasted_iota(jnp.int32, (N, NG), 1)).astype(f32)
    cnt16 = jnp.sum(oh, axis=0, keepdims=True)
    ohn = oh * (1.0 / jnp.maximum(cnt16, 1.0))
    mean16 = lax.dot_general(ohn, x_new, (((0,), (0,)), ((), ())),
                             preferred_element_type=f32)
    u = u_ref[...]
    g0 = jax.nn.gelu(jnp.dot(u, w0gu_ref[...], preferred_element_type=f32)
                     + jnp.dot(mean16, w0gm_ref[...], preferred_element_type=f32)
                     + b0g_ref[...])
    g1 = jax.nn.gelu(jnp.dot(g0, w1g_ref[...], preferred_element_type=f32) + b1g_ref[...])
    unew_ref[...] = _ln(g1 + g0, sg_ref[...], bg_ref[...])


def _post_call(sa, sb, cvec, pu2, b16, u, w0n2a, w1n2, b1n2, sn2, bn2,
               w0gu, w0gm, b0g, w1g, b1g, sg, bg):
    full = lambda shape: pl.BlockSpec(shape, lambda: (0,) * len(shape))
    return pl.pallas_call(
        _post_body,
        in_specs=[
            full((NP, H)), full((NP, H)), full((NP, 1)), full((N, H)), full((N, NG)),
            full((NG, DU)),
            full((H, H)), full((H, H)), full((1, H)), full((1, H)), full((1, H)),
            full((DU, H)), full((H, H)), full((1, H)), full((H, H)), full((1, H)),
            full((1, H)), full((1, H)),
        ],
        out_specs=[full((N, H)), full((NG, H))],
        out_shape=[
            jax.ShapeDtypeStruct((N, H), f32),
            jax.ShapeDtypeStruct((NG, H), f32),
        ],
    )(sa, sb, cvec, pu2, b16, u, w0n2a, w1n2, b1n2, sn2, bn2,
      w0gu, w0gm, b0g, w1g, b1g, sg, bg)


def kernel(x, edge_index, edge_attr, u, batch, params):
    pe, pn1, pn2, pg = params["edge"], params["node1"], params["node2"], params["glob"]
    row = edge_index[0]
    col = edge_index[1]
    b16 = jnp.broadcast_to(batch[:, None], (N, NG))
    zrow = jnp.zeros((ZR, H), f32)

    r1 = lambda v: v.reshape(1, H)
    w0e = pe["W0"]
    pre_row, pre_col, pre_u2 = _pre_call(
        x, b16, u,
        w0e[0:D], w0e[D:2 * D], w0e[2 * D + DE:], r1(pe["b0"]),
        pn1["W0"][0:D], r1(pn1["b0"]),
        pn2["W0"][H:], r1(pn2["b0"]),
    )

    edge_w = (w0e[2 * D:2 * D + DE], pe["W1"], r1(pe["b1"]), r1(pe["ln_s"]),
              r1(pe["ln_b"]), pn1["W0"][D:], pn1["W1"], r1(pn1["b1"]),
              r1(pn1["ln_s"]), r1(pn1["ln_b"]))

    rowA, rowB = row[:E1], row[E1:]
    colA, colB = col[:E1], col[E1:]

    grA, gcA = _sc_gather(pre_row, pre_col, rowA, colA, E1)
    eA, nhA, cntA = _edge_call(grA, gcA, edge_attr[:E1], rowA.reshape(E1, 1),
                               *edge_w, ne=E1)
    grB, gcB = _sc_gather(pre_row, pre_col, rowB, colB, E2)
    sumsA = _sc_sums(nhA, rowA, zrow, E1)
    eB, nhB, cntB = _edge_call(grB, gcB, edge_attr[E1:], rowB.reshape(E2, 1),
                               *edge_w, ne=E2)
    sumsB = _sc_sums(nhB, rowB, zrow, E2)
    e_new = jnp.concatenate([eA, eB], axis=0)

    cvec = (cntA + cntB).reshape(NP, 1)
    x_new, u_new = _post_call(
        sumsA, sumsB, cvec, pre_u2, b16, u,
        pn2["W0"][0:H], pn2["W1"], r1(pn2["b1"]), r1(pn2["ln_s"]), r1(pn2["ln_b"]),
        pg["W0"][0:DU], pg["W0"][DU:], r1(pg["b0"]), pg["W1"], r1(pg["b1"]),
        r1(pg["ln_s"]), r1(pg["ln_b"]),
    )
    return x_new, e_new, u_new


# speedup vs baseline: 1.0290x; 1.0290x over previous
import functools

import jax
import jax.numpy as jnp
from jax import lax
from jax.experimental import pallas as pl
from jax.experimental.pallas import tpu as pltpu
from jax.experimental.pallas import tpu_sc as plsc

N = 10000
E = 320000
D = 128
DE = 16
DU = 128
H = 128
NG = 16

E1 = 166400
E2 = E - E1
CH = 80
GCH = 40
NP = 10240
NHI = NP // 128
EB = 1600
NB = 2000

ZR = 128
HALF = NP // 2
ACC = 6144
TRASH = HALF

f32 = jnp.float32
bf16 = jnp.bfloat16


def _pack_pair(a):
    u = lax.bitcast_convert_type(a.astype(bf16), jnp.uint16).astype(jnp.uint32)
    w = a.shape[1]
    return u[:, : w // 2] | (u[:, w // 2:] << 16)


def _unpack_lo(p):
    return lax.bitcast_convert_type(p << 16, f32)


def _unpack_hi(p):
    return lax.bitcast_convert_type(p & jnp.uint32(0xFFFF0000), f32)


def _ln(h, s, b):
    m = jnp.mean(h, axis=-1, keepdims=True)
    v = jnp.mean((h - m) ** 2, axis=-1, keepdims=True)
    return (h - m) * lax.rsqrt(v + 1e-5) * s + b


def _pre_body(x_ref, b16_ref, u_ref, wsrc_ref, wdst_ref, wu_ref, b0e_ref,
              wn1d_ref, b0n1_ref, wn2u_ref, b0n2_ref,
              prow_ref, pcol_ref, pu2_ref):
    oh = (b16_ref[...] == lax.broadcasted_iota(jnp.int32, (NB, NG), 1)).astype(f32)
    u = u_ref[...]
    x = x_ref[...]
    uproj_e = jnp.dot(u, wu_ref[...], preferred_element_type=f32)
    prow = (jnp.dot(x, wsrc_ref[...], preferred_element_type=f32)
            + jnp.dot(oh, uproj_e, preferred_element_type=f32)
            + b0e_ref[...])
    col_a = jnp.dot(x, wdst_ref[...], preferred_element_type=f32)
    col_b = jnp.dot(x, wn1d_ref[...], preferred_element_type=f32) + b0n1_ref[...]
    prow_ref[...] = prow
    pcol_ref[...] = _pack_pair(jnp.concatenate([col_a, col_b], axis=1))
    pu2_ref[...] = (jnp.dot(oh, jnp.dot(u, wn2u_ref[...], preferred_element_type=f32),
                            preferred_element_type=f32) + b0n2_ref[...])


def _pre_call(x, b16, u, wsrc, wdst, wu, b0e, wn1d, b0n1, wn2u, b0n2):
    full = lambda shape: pl.BlockSpec(shape, lambda i: (0,) * len(shape))
    return pl.pallas_call(
        _pre_body,
        grid=(N // NB,),
        in_specs=[
            pl.BlockSpec((NB, D), lambda i: (i, 0)),
            pl.BlockSpec((NB, NG), lambda i: (i, 0)),
            full((NG, DU)), full((D, H)), full((D, H)), full((DU, H)),
            full((1, H)), full((D, H)), full((1, H)), full((DU, H)), full((1, H)),
        ],
        out_specs=[
            pl.BlockSpec((NB, H), lambda i: (i, 0)),
            pl.BlockSpec((NB, H), lambda i: (i, 0)),
            pl.BlockSpec((NB, H), lambda i: (i, 0)),
        ],
        out_shape=[
            jax.ShapeDtypeStruct((N, H), f32),
            jax.ShapeDtypeStruct((N, H), jnp.uint32),
            jax.ShapeDtypeStruct((N, H), f32),
        ],
    )(x, b16, u, wsrc, wdst, wu, b0e, wn1d, b0n1, wn2u, b0n2)


def _sc_gather(pre_row, pre_col, row, col, ne):
    perw = ne // 32
    nch = perw // GCH
    npair = nch // 2 - 1
    mesh = plsc.VectorSubcoreMesh(core_axis_name="c", subcore_axis_name="s")

    @functools.partial(
        pl.kernel,
        out_type=(jax.ShapeDtypeStruct((ne, H), f32),
                  jax.ShapeDtypeStruct((ne, H), jnp.uint32)),
        mesh=mesh,
        scratch_types=[
            pltpu.VMEM((perw,), jnp.int32),
            pltpu.VMEM((perw,), jnp.int32),
            pltpu.VMEM((GCH, H), f32),
            pltpu.VMEM((GCH, H), f32),
            pltpu.VMEM((GCH, H), jnp.uint32),
            pltpu.VMEM((GCH, H), jnp.uint32),
        ] + [pltpu.SemaphoreType.DMA] * 8,
    )
    def gk(prer_hbm, prec_hbm, row_hbm, col_hbm, outr_hbm, outc_hbm,
           ridx, cidx, rb0, rb1, cb0, cb1, sr0, sr1, sc0, sc1, wr0, wr1, wc0, wc1):
        wid = lax.axis_index("s") * 2 + lax.axis_index("c")
        base = wid * perw
        pltpu.sync_copy(row_hbm.at[pl.ds(base, perw)], ridx)
        pltpu.sync_copy(col_hbm.at[pl.ds(base, perw)], cidx)
        rbufs, cbufs = (rb0, rb1), (cb0, cb1)
        srs, scs = (sr0, sr1), (sc0, sc1)
        wrs, wcs = (wr0, wr1), (wc0, wc1)

        def issue_g(j, s):
            pltpu.async_copy(prer_hbm.at[ridx.at[pl.ds(j * GCH, GCH)]], rbufs[s], srs[s])
            pltpu.async_copy(prec_hbm.at[cidx.at[pl.ds(j * GCH, GCH)]], cbufs[s], scs[s])

        def wait_g(s):
            pltpu.make_async_copy(prer_hbm.at[ridx.at[pl.ds(0, GCH)]], rbufs[s], srs[s]).wait()
            pltpu.make_async_copy(prec_hbm.at[cidx.at[pl.ds(0, GCH)]], cbufs[s], scs[s]).wait()

        def start_wb(j, s):
            pltpu.async_copy(rbufs[s], outr_hbm.at[pl.ds(base + j * GCH, GCH)], wrs[s])
            pltpu.async_copy(cbufs[s], outc_hbm.at[pl.ds(base + j * GCH, GCH)], wcs[s])

        def wait_wb(s):
            pltpu.make_async_copy(rbufs[s], outr_hbm.at[pl.ds(base, GCH)], wrs[s]).wait()
            pltpu.make_async_copy(cbufs[s], outc_hbm.at[pl.ds(base, GCH)], wcs[s]).wait()

        issue_g(0, 0)
        issue_g(1, 1)

        def body(jj, carry):
            j0 = jj * 2
            wait_g(0)
            start_wb(j0, 0)
            wait_g(1)
            start_wb(j0 + 1, 1)
            wait_wb(0)
            issue_g(j0 + 2, 0)
            wait_wb(1)
            issue_g(j0 + 3, 1)
            return carry

        lax.fori_loop(0, npair, body, 0)
        wait_g(0)
        start_wb(nch - 2, 0)
        wait_g(1)
        start_wb(nch - 1, 1)
        wait_wb(0)
        wait_wb(1)

    return gk(pre_row, pre_col, row, col)


def _edge_body(gr_ref, gc_ref, ea_ref, w0ea_ref, w1e_ref, b1e_ref,
               se_ref, be_ref, w0n1e_ref, w1n1_ref, b1n1_ref, sn1_ref, bn1_ref,
               enew_ref, nh_ref):
    gcp = gc_ref[...]
    gca = _unpack_lo(gcp)
    gcb = _unpack_hi(gcp)
    eap = jnp.dot(ea_ref[...], w0ea_ref[...], preferred_element_type=f32)
    h0 = jax.nn.gelu(gr_ref[...] + gca + eap)
    h1 = jax.nn.gelu(jnp.dot(h0, w1e_ref[...], preferred_element_type=f32) + b1e_ref[...])
    e_new = _ln(h1 + h0, se_ref[...], be_ref[...])
    m0 = jax.nn.gelu(gcb + jnp.dot(e_new, w0n1e_ref[...], preferred_element_type=f32))
    m1 = jax.nn.gelu(jnp.dot(m0, w1n1_ref[...], preferred_element_type=f32) + b1n1_ref[...])
    enew_ref[...] = e_new
    nh_ref[...] = _ln(m1 + m0, sn1_ref[...], bn1_ref[...])


def _edge_call(gr, gc, edge_attr, w0ea, w1e, b1e, se, be,
               w0n1e, w1n1, b1n1, sn1, bn1, ne):
    full = lambda shape: pl.BlockSpec(shape, lambda i: (0,) * len(shape))
    return pl.pallas_call(
        _edge_body,
        grid=(ne // EB,),
        in_specs=[
            pl.BlockSpec((EB, H), lambda i: (i, 0)),
            pl.BlockSpec((EB, H), lambda i: (i, 0)),
            pl.BlockSpec((EB, DE), lambda i: (i, 0)),
            full((DE, H)), full((H, H)), full((1, H)), full((1, H)), full((1, H)),
            full((H, H)), full((H, H)), full((1, H)), full((1, H)), full((1, H)),
        ],
        out_specs=[
            pl.BlockSpec((EB, H), lambda i: (i, 0)),
            pl.BlockSpec((EB, H), lambda i: (i, 0)),
        ],
        out_shape=[
            jax.ShapeDtypeStruct((ne, H), f32),
            jax.ShapeDtypeStruct((ne, H), f32),
        ],
    )(gr, gc, edge_attr, w0ea, w1e, b1e, se, be, w0n1e, w1n1, b1n1, sn1, bn1)


CB = 2000


def _cnt_body(row_ref, cnt_ref):
    i = pl.program_id(0)

    @pl.when(i == 0)
    def _():
        cnt_ref[...] = jnp.zeros((NHI, 128), f32)

    r2 = row_ref[...]
    oh_hi = ((r2 >> 7) == lax.broadcasted_iota(jnp.int32, (CB, NHI), 1)).astype(f32)
    oh_lo = ((r2 & 127) == lax.broadcasted_iota(jnp.int32, (CB, 128), 1)).astype(f32)
    cnt_ref[...] += lax.dot_general(oh_hi, oh_lo, (((0,), (0,)), ((), ())),
                                    preferred_element_type=f32)


def _cnt_call(row2d):
    return pl.pallas_call(
        _cnt_body,
        grid=(E // CB,),
        in_specs=[pl.BlockSpec((CB, 1), lambda i: (i, 0))],
        out_specs=pl.BlockSpec((NHI, 128), lambda i: (0, 0)),
        out_shape=jax.ShapeDtypeStruct((NHI, 128), f32),
    )(row2d)


def _sc_sums(nh, row, zrow, ne):
    pert = ne // 16
    nchs = pert // CH
    mesh = plsc.VectorSubcoreMesh(core_axis_name="c", subcore_axis_name="s")

    @functools.partial(
        pl.kernel,
        out_type=jax.ShapeDtypeStruct((NP, H), f32),
        mesh=mesh,
        scratch_types=[
            pltpu.VMEM((pert,), jnp.int32),
            pltpu.VMEM((CH,), jnp.int32),
            pltpu.VMEM((CH, H), f32),
            pltpu.VMEM((CH, H), f32),
            pltpu.VMEM((ZR, H), f32),
            pltpu.VMEM_SHARED((ACC, H), f32),
            pltpu.SemaphoreType.DMA,
            pltpu.SemaphoreType.DMA,
        ],
    )
    def sk(nh_hbm, row_hbm, zrow_hbm, outs_hbm,
           idxall, idxb2, vb0, vb1, zrb, acc_sh, sl0, sl1):
        cid = lax.axis_index("c")
        sid = lax.axis_index("s")
        base = sid * pert
        lo = cid * HALF
        pltpu.sync_copy(zrow_hbm, zrb)
        pltpu.sync_copy(row_hbm.at[pl.ds(base, pert)], idxall)

        def zbody(k, carry):
            pltpu.sync_copy(zrb, acc_sh.at[pl.ds(sid * (ACC // 16) + k * ZR, ZR)])
            return carry

        lax.fori_loop(0, ACC // 16 // ZR, zbody, 0)
        plsc.subcore_barrier()
        vbufs, sls = (vb0, vb1), (sl0, sl1)

        def load(j, s):
            pltpu.async_copy(nh_hbm.at[pl.ds(base + j * CH, CH)], vbufs[s], sls[s])

        def wait_load(s):
            pltpu.make_async_copy(nh_hbm.at[pl.ds(base, CH)], vbufs[s], sls[s]).wait()

        def remap(j):
            for k in range(CH // 16):
                v = idxall[pl.ds(j * CH + k * 16, 16)] - lo
                oob = (v < 0) | (v >= HALF)
                idxb2[pl.ds(k * 16, 16)] = jnp.where(oob, TRASH, v)

        def scat(s):
            pltpu.sync_copy(vbufs[s], acc_sh.at[idxb2], add=True)

        load(0, 0)

        def body(jj, carry):
            j0 = jj * 2
            wait_load(0)
            load(j0 + 1, 1)
            remap(j0)
            scat(0)
            wait_load(1)
            load(j0 + 2, 0)
            remap(j0 + 1)
            scat(1)
            return carry

        lax.fori_loop(0, nchs // 2 - 1, body, 0)
        wait_load(0)
        load(nchs - 1, 1)
        remap(nchs - 2)
        scat(0)
        wait_load(1)
        remap(nchs - 1)
        scat(1)
        plsc.subcore_barrier()

        def dbody(k, carry):
            src = sid * (HALF // 16) + k * CH
            pltpu.sync_copy(acc_sh.at[pl.ds(src, CH)], vb0)
            pltpu.sync_copy(vb0, outs_hbm.at[pl.ds(cid * HALF + src, CH)])
            return carry

        lax.fori_loop(0, HALF // 16 // CH, dbody, 0)

    return sk(nh, row, zrow)


def _post_body(sa_ref, sb_ref, cnt_ref, pu2_ref, b16_ref, u_ref,
               w0n2a_ref, w1n2_ref, b1n2_ref, sn2_ref, bn2_ref,
               w0gu_ref, w0gm_ref, b0g_ref, w1g_ref, b1g_ref, sg_ref, bg_ref,
               xnew_ref, unew_ref):
    s = sa_ref[:N, :] + sb_ref[:N, :]
    c = cnt_ref[:N, :]
    agg = s * (1.0 / jnp.maximum(c, 1.0))
    a0 = jax.nn.gelu(jnp.dot(agg, w0n2a_ref[...], preferred_element_type=f32) + pu2_ref[...])
    a1 = jax.nn.gelu(jnp.dot(a0, w1n2_ref[...], preferred_element_type=f32) + b1n2_ref[...])
    x_new = _ln(a1 + a0, sn2_ref[...], bn2_ref[...])
    xnew_ref[...] = x_new
    oh = (b16_ref[...] == lax.broadcasted_iota(jnp.int32, (N, NG), 1)).astype(f32)
    cnt16 = jnp.sum(oh, axis=0, keepdims=True)
    ohn = oh * (1.0 / jnp.maximum(cnt16, 1.0))
    mean16 = lax.dot_general(ohn, x_new, (((0,), (0,)), ((), ())),
                             preferred_element_type=f32)
    u = u_ref[...]
    g0 = jax.nn.gelu(jnp.dot(u, w0gu_ref[...], preferred_element_type=f32)
                     + jnp.dot(mean16, w0gm_ref[...], preferred_element_type=f32)
                     + b0g_ref[...])
    g1 = jax.nn.gelu(jnp.dot(g0, w1g_ref[...], preferred_element_type=f32) + b1g_ref[...])
    unew_ref[...] = _ln(g1 + g0, sg_ref[...], bg_ref[...])


def _post_call(sa, sb, cvec, pu2, b16, u, w0n2a, w1n2, b1n2, sn2, bn2,
               w0gu, w0gm, b0g, w1g, b1g, sg, bg):
    full = lambda shape: pl.BlockSpec(shape, lambda: (0,) * len(shape))
    return pl.pallas_call(
        _post_body,
        in_specs=[
            full((NP, H)), full((NP, H)), full((NP, 1)), full((N, H)), full((N, NG)),
            full((NG, DU)),
            full((H, H)), full((H, H)), full((1, H)), full((1, H)), full((1, H)),
            full((DU, H)), full((H, H)), full((1, H)), full((H, H)), full((1, H)),
            full((1, H)), full((1, H)),
        ],
        out_specs=[full((N, H)), full((NG, H))],
        out_shape=[
            jax.ShapeDtypeStruct((N, H), f32),
            jax.ShapeDtypeStruct((NG, H), f32),
        ],
    )(sa, sb, cvec, pu2, b16, u, w0n2a, w1n2, b1n2, sn2, bn2,
      w0gu, w0gm, b0g, w1g, b1g, sg, bg)


def kernel(x, edge_index, edge_attr, u, batch, params):
    pe, pn1, pn2, pg = params["edge"], params["node1"], params["node2"], params["glob"]
    row = edge_index[0]
    col = edge_index[1]
    b16 = jnp.broadcast_to(batch[:, None], (N, NG))
    zrow = jnp.zeros((ZR, H), f32)

    r1 = lambda v: v.reshape(1, H)
    w0e = pe["W0"]
    pre_row, pre_col, pre_u2 = _pre_call(
        x, b16, u,
        w0e[0:D], w0e[D:2 * D], w0e[2 * D + DE:], r1(pe["b0"]),
        pn1["W0"][0:D], r1(pn1["b0"]),
        pn2["W0"][H:], r1(pn2["b0"]),
    )

    edge_w = (w0e[2 * D:2 * D + DE], pe["W1"], r1(pe["b1"]), r1(pe["ln_s"]),
              r1(pe["ln_b"]), pn1["W0"][D:], pn1["W1"], r1(pn1["b1"]),
              r1(pn1["ln_s"]), r1(pn1["ln_b"]))

    rowA, rowB = row[:E1], row[E1:]
    colA, colB = col[:E1], col[E1:]

    cnt2d = _cnt_call(row.reshape(E, 1))
    grA, gcA = _sc_gather(pre_row, pre_col, rowA, colA, E1)
    eA, nhA = _edge_call(grA, gcA, edge_attr[:E1], *edge_w, ne=E1)
    grB, gcB = _sc_gather(pre_row, pre_col, rowB, colB, E2)
    sumsA = _sc_sums(nhA, rowA, zrow, E1)
    eB, nhB = _edge_call(grB, gcB, edge_attr[E1:], *edge_w, ne=E2)
    sumsB = _sc_sums(nhB, rowB, zrow, E2)
    e_new = jnp.concatenate([eA, eB], axis=0)

    cvec = cnt2d.reshape(NP, 1)
    x_new, u_new = _post_call(
        sumsA, sumsB, cvec, pre_u2, b16, u,
        pn2["W0"][0:H], pn2["W1"], r1(pn2["b1"]), r1(pn2["ln_s"]), r1(pn2["ln_b"]),
        pg["W0"][0:DU], pg["W0"][DU:], r1(pg["b0"]), pg["W1"], r1(pg["b1"]),
        r1(pg["ln_s"]), r1(pg["ln_b"]),
    )
    return x_new, e_new, u_new


# speedup vs baseline: 1.1214x; 1.0899x over previous
import functools

import jax
import jax.numpy as jnp
from jax import lax
from jax.experimental import pallas as pl
from jax.experimental.pallas import tpu as pltpu
from jax.experimental.pallas import tpu_sc as plsc

N = 10000
E = 320000
D = 128
DE = 16
DU = 128
H = 128
NG = 16

NW = 32
PERW = E // NW
CH = 80
NP = 10240
NPT = NP // 16
EB = 2000
NB = 2000

f32 = jnp.float32
bf16 = jnp.bfloat16


def _pack_pair(a):
    u = lax.bitcast_convert_type(a.astype(bf16), jnp.uint16).astype(jnp.uint32)
    w = a.shape[1]
    return u[:, : w // 2] | (u[:, w // 2:] << 16)


def _unpack_lo(p):
    return lax.bitcast_convert_type(p << 16, f32)


def _unpack_hi(p):
    return lax.bitcast_convert_type(p & jnp.uint32(0xFFFF0000), f32)


def _ln(h, s, b):
    m = jnp.mean(h, axis=-1, keepdims=True)
    v = jnp.mean((h - m) ** 2, axis=-1, keepdims=True)
    return (h - m) * lax.rsqrt(v + 1e-5) * s + b


def _pre_body(x_ref, b16_ref, u_ref, wsrc_ref, wdst_ref, wu_ref, b0e_ref,
              wn1d_ref, b0n1_ref, wn2u_ref, b0n2_ref,
              prow_ref, pcol_ref, pu2_ref):
    oh = (b16_ref[...] == lax.broadcasted_iota(jnp.int32, (NB, NG), 1)).astype(f32)
    u = u_ref[...]
    x = x_ref[...]
    uproj_e = jnp.dot(u, wu_ref[...], preferred_element_type=f32)
    prow = (jnp.dot(x, wsrc_ref[...], preferred_element_type=f32)
            + jnp.dot(oh, uproj_e, preferred_element_type=f32)
            + b0e_ref[...])
    col_a = jnp.dot(x, wdst_ref[...], preferred_element_type=f32)
    col_b = jnp.dot(x, wn1d_ref[...], preferred_element_type=f32) + b0n1_ref[...]
    prow_ref[...] = prow
    pcol_ref[...] = _pack_pair(jnp.concatenate([col_a, col_b], axis=1))
    pu2_ref[...] = (jnp.dot(oh, jnp.dot(u, wn2u_ref[...], preferred_element_type=f32),
                            preferred_element_type=f32) + b0n2_ref[...])


def _pre_call(x, b16, u, wsrc, wdst, wu, b0e, wn1d, b0n1, wn2u, b0n2):
    full = lambda shape: pl.BlockSpec(shape, lambda i: (0,) * len(shape))
    return pl.pallas_call(
        _pre_body,
        grid=(N // NB,),
        in_specs=[
            pl.BlockSpec((NB, D), lambda i: (i, 0)),
            pl.BlockSpec((NB, NG), lambda i: (i, 0)),
            full((NG, DU)), full((D, H)), full((D, H)), full((DU, H)),
            full((1, H)), full((D, H)), full((1, H)), full((DU, H)), full((1, H)),
        ],
        out_specs=[
            pl.BlockSpec((NB, H), lambda i: (i, 0)),
            pl.BlockSpec((NB, H), lambda i: (i, 0)),
            pl.BlockSpec((NB, H), lambda i: (i, 0)),
        ],
        out_shape=[
            jax.ShapeDtypeStruct((N, H), f32),
            jax.ShapeDtypeStruct((N, H), jnp.uint32),
            jax.ShapeDtypeStruct((N, H), f32),
        ],
    )(x, b16, u, wsrc, wdst, wu, b0e, wn1d, b0n1, wn2u, b0n2)


GCH = 40
NCH = PERW // GCH
NPAIR = NCH // 2 - 1


def _sc_gather(pre_row, pre_col, row, col):
    mesh = plsc.VectorSubcoreMesh(core_axis_name="c", subcore_axis_name="s")

    @functools.partial(
        pl.kernel,
        out_type=(jax.ShapeDtypeStruct((E, H), f32),
                  jax.ShapeDtypeStruct((E, H), jnp.uint32)),
        mesh=mesh,
        scratch_types=[
            pltpu.VMEM((PERW,), jnp.int32),
            pltpu.VMEM((PERW,), jnp.int32),
            pltpu.VMEM((GCH, H), f32),
            pltpu.VMEM((GCH, H), f32),
            pltpu.VMEM((GCH, H), jnp.uint32),
            pltpu.VMEM((GCH, H), jnp.uint32),
        ] + [pltpu.SemaphoreType.DMA] * 8,
    )
    def gk(prer_hbm, prec_hbm, row_hbm, col_hbm, outr_hbm, outc_hbm,
           ridx, cidx, rb0, rb1, cb0, cb1, sr0, sr1, sc0, sc1, wr0, wr1, wc0, wc1):
        wid = lax.axis_index("s") * 2 + lax.axis_index("c")
        base = wid * PERW
        pltpu.sync_copy(row_hbm.at[pl.ds(base, PERW)], ridx)
        pltpu.sync_copy(col_hbm.at[pl.ds(base, PERW)], cidx)
        rbufs, cbufs = (rb0, rb1), (cb0, cb1)
        srs, scs = (sr0, sr1), (sc0, sc1)
        wrs, wcs = (wr0, wr1), (wc0, wc1)

        def issue_g(j, s):
            pltpu.async_copy(prer_hbm.at[ridx.at[pl.ds(j * GCH, GCH)]], rbufs[s], srs[s])
            pltpu.async_copy(prec_hbm.at[cidx.at[pl.ds(j * GCH, GCH)]], cbufs[s], scs[s])

        def wait_g(s):
            pltpu.make_async_copy(prer_hbm.at[ridx.at[pl.ds(0, GCH)]], rbufs[s], srs[s]).wait()
            pltpu.make_async_copy(prec_hbm.at[cidx.at[pl.ds(0, GCH)]], cbufs[s], scs[s]).wait()

        def start_wb(j, s):
            pltpu.async_copy(rbufs[s], outr_hbm.at[pl.ds(base + j * GCH, GCH)], wrs[s])
            pltpu.async_copy(cbufs[s], outc_hbm.at[pl.ds(base + j * GCH, GCH)], wcs[s])

        def wait_wb(s):
            pltpu.make_async_copy(rbufs[s], outr_hbm.at[pl.ds(base, GCH)], wrs[s]).wait()
            pltpu.make_async_copy(cbufs[s], outc_hbm.at[pl.ds(base, GCH)], wcs[s]).wait()

        issue_g(0, 0)
        issue_g(1, 1)

        def body(jj, carry):
            j0 = jj * 2
            wait_g(0)
            start_wb(j0, 0)
            wait_g(1)
            start_wb(j0 + 1, 1)
            wait_wb(0)
            issue_g(j0 + 2, 0)
            wait_wb(1)
            issue_g(j0 + 3, 1)
            return carry

        lax.fori_loop(0, NPAIR, body, 0)
        wait_g(0)
        start_wb(NCH - 2, 0)
        wait_g(1)
        start_wb(NCH - 1, 1)
        wait_wb(0)
        wait_wb(1)

    return gk(pre_row, pre_col, row, col)


def _edge_body(gr_ref, gc_ref, ea_ref, w0ea_ref, w1e_ref, b1e_ref, se_ref, be_ref,
               w0n1e_ref, w1n1_ref, b1n1_ref, sn1_ref, bn1_ref,
               enew_ref, nh_ref):
    gcp = gc_ref[...]
    gca = _unpack_lo(gcp)
    gcb = _unpack_hi(gcp)
    eap = jnp.dot(ea_ref[...], w0ea_ref[...], preferred_element_type=f32)
    h0 = jax.nn.gelu(gr_ref[...] + gca + eap)
    h1 = jax.nn.gelu(jnp.dot(h0, w1e_ref[...], preferred_element_type=f32) + b1e_ref[...])
    e_new = _ln(h1 + h0, se_ref[...], be_ref[...])
    m0 = jax.nn.gelu(gcb + jnp.dot(e_new, w0n1e_ref[...], preferred_element_type=f32))
    m1 = jax.nn.gelu(jnp.dot(m0, w1n1_ref[...], preferred_element_type=f32) + b1n1_ref[...])
    enew_ref[...] = e_new
    nh_ref[...] = _ln(m1 + m0, sn1_ref[...], bn1_ref[...])


def _edge_call(gr, gc, edge_attr, w0ea, w1e, b1e, se, be, w0n1e, w1n1, b1n1, sn1, bn1):
    full = lambda shape: pl.BlockSpec(shape, lambda i: (0,) * len(shape))
    return pl.pallas_call(
        _edge_body,
        grid=(E // EB,),
        in_specs=[
            pl.BlockSpec((EB, H), lambda i: (i, 0)),
            pl.BlockSpec((EB, H), lambda i: (i, 0)),
            pl.BlockSpec((EB, DE), lambda i: (i, 0)),
            full((DE, H)), full((H, H)), full((1, H)), full((1, H)), full((1, H)),
            full((H, H)), full((H, H)), full((1, H)), full((1, H)), full((1, H)),
        ],
        out_specs=[
            pl.BlockSpec((EB, H), lambda i: (i, 0)),
            pl.BlockSpec((EB, H), lambda i: (i, 0)),
        ],
        out_shape=[
            jax.ShapeDtypeStruct((E, H), f32),
            jax.ShapeDtypeStruct((E, H), f32),
        ],
    )(gr, gc, edge_attr, w0ea, w1e, b1e, se, be, w0n1e, w1n1, b1n1, sn1, bn1)


ZR = 128
HALF = NP // 2
ACC = 6144
TRASH = HALF
PERT = E // 16


NCHS = PERT // CH


def _scatter_common(cid, sid, idxall, idxb2, vbuf, zrb, acc_sh):
    lo = cid * HALF

    def zero_acc():
        def zbody(k, carry):
            pltpu.sync_copy(zrb, acc_sh.at[pl.ds(sid * (ACC // 16) + k * ZR, ZR)])
            return carry

        lax.fori_loop(0, ACC // 16 // ZR, zbody, 0)

    def remap(j):
        for k in range(CH // 16):
            v = idxall[pl.ds(j * CH + k * 16, 16)] - lo
            oob = (v < 0) | (v >= HALF)
            idxb2[pl.ds(k * 16, 16)] = jnp.where(oob, TRASH, v)

    def drain(dst_hbm):
        def dbody(k, carry):
            src = sid * (HALF // 16) + k * CH
            pltpu.sync_copy(acc_sh.at[pl.ds(src, CH)], vbuf)
            pltpu.sync_copy(vbuf, dst_hbm.at[pl.ds(cid * HALF + src, CH)])
            return carry

        lax.fori_loop(0, HALF // 16 // CH, dbody, 0)

    return zero_acc, remap, drain


def _sc_sums(nh, row, zrow):
    mesh = plsc.VectorSubcoreMesh(core_axis_name="c", subcore_axis_name="s")

    @functools.partial(
        pl.kernel,
        out_type=jax.ShapeDtypeStruct((NP, H), f32),
        mesh=mesh,
        scratch_types=[
            pltpu.VMEM((PERT,), jnp.int32),
            pltpu.VMEM((CH,), jnp.int32),
            pltpu.VMEM((CH, H), f32),
            pltpu.VMEM((CH, H), f32),
            pltpu.VMEM((ZR, H), f32),
            pltpu.VMEM_SHARED((ACC, H), f32),
            pltpu.SemaphoreType.DMA,
            pltpu.SemaphoreType.DMA,
        ],
    )
    def sk(nh_hbm, row_hbm, zrow_hbm, outs_hbm,
           idxall, idxb2, vb0, vb1, zrb, acc_sh, sl0, sl1):
        cid = lax.axis_index("c")
        sid = lax.axis_index("s")
        base = sid * PERT
        pltpu.sync_copy(zrow_hbm, zrb)
        pltpu.sync_copy(row_hbm.at[pl.ds(base, PERT)], idxall)
        zero_acc, remap, drain = _scatter_common(cid, sid, idxall, idxb2, vb0, zrb, acc_sh)
        zero_acc()
        plsc.subcore_barrier()
        vbufs, sls = (vb0, vb1), (sl0, sl1)

        def load(j, s):
            pltpu.async_copy(nh_hbm.at[pl.ds(base + j * CH, CH)], vbufs[s], sls[s])

        def wait_load(s):
            pltpu.make_async_copy(nh_hbm.at[pl.ds(base, CH)], vbufs[s], sls[s]).wait()

        def scat(s):
            pltpu.sync_copy(vbufs[s], acc_sh.at[idxb2], add=True)

        load(0, 0)

        def body(jj, carry):
            j0 = jj * 2
            wait_load(0)
            load(j0 + 1, 1)
            remap(j0)
            scat(0)
            wait_load(1)
            load(j0 + 2, 0)
            remap(j0 + 1)
            scat(1)
            return carry

        lax.fori_loop(0, NCHS // 2 - 1, body, 0)
        wait_load(0)
        load(NCHS - 1, 1)
        remap(NCHS - 2)
        scat(0)
        wait_load(1)
        remap(NCHS - 1)
        scat(1)
        plsc.subcore_barrier()
        drain(outs_hbm)

    return sk(nh, row, zrow)


def _sc_counts(row, zrow, ones_rows):
    mesh = plsc.VectorSubcoreMesh(core_axis_name="c", subcore_axis_name="s")

    @functools.partial(
        pl.kernel,
        out_type=jax.ShapeDtypeStruct((NP, H), f32),
        mesh=mesh,
        scratch_types=[
            pltpu.VMEM((PERT,), jnp.int32),
            pltpu.VMEM((CH,), jnp.int32),
            pltpu.VMEM((CH, H), f32),
            pltpu.VMEM((ZR, H), f32),
            pltpu.VMEM_SHARED((ACC, H), f32),
            pltpu.VMEM((CH, H), f32),
        ],
    )
    def ck(row_hbm, zrow_hbm, ones_hbm, outc_hbm,
           idxall, idxb2, vbuf, zrb, acc_sh, onesb):
        cid = lax.axis_index("c")
        sid = lax.axis_index("s")
        pltpu.sync_copy(zrow_hbm, zrb)
        pltpu.sync_copy(ones_hbm, onesb)
        pltpu.sync_copy(row_hbm.at[pl.ds(sid * PERT, PERT)], idxall)
        zero_acc, remap, drain = _scatter_common(cid, sid, idxall, idxb2, vbuf, zrb, acc_sh)
        zero_acc()
        plsc.subcore_barrier()

        def body(j, carry):
            remap(j)
            pltpu.sync_copy(onesb, acc_sh.at[idxb2], add=True)
            return carry

        lax.fori_loop(0, NCHS, body, 0)
        plsc.subcore_barrier()
        drain(outc_hbm)

    return ck(row, zrow, ones_rows)


def _post_body(sums_ref, cnts_ref, pu2_ref, b16_ref, u_ref,
               w0n2a_ref, w1n2_ref, b1n2_ref, sn2_ref, bn2_ref,
               w0gu_ref, w0gm_ref, b0g_ref, w1g_ref, b1g_ref, sg_ref, bg_ref,
               xnew_ref, unew_ref):
    s = sums_ref[:N, :]
    c = cnts_ref[:N, 0:1]
    agg = s * (1.0 / jnp.maximum(c, 1.0))
    a0 = jax.nn.gelu(jnp.dot(agg, w0n2a_ref[...], preferred_element_type=f32) + pu2_ref[...])
    a1 = jax.nn.gelu(jnp.dot(a0, w1n2_ref[...], preferred_element_type=f32) + b1n2_ref[...])
    x_new = _ln(a1 + a0, sn2_ref[...], bn2_ref[...])
    xnew_ref[...] = x_new
    oh = (b16_ref[...] == lax.broadcasted_iota(jnp.int32, (N, NG), 1)).astype(f32)
    cnt16 = jnp.sum(oh, axis=0, keepdims=True)
    ohn = oh * (1.0 / jnp.maximum(cnt16, 1.0))
    mean16 = lax.dot_general(ohn, x_new, (((0,), (0,)), ((), ())),
                             preferred_element_type=f32)
    u = u_ref[...]
    g0 = jax.nn.gelu(jnp.dot(u, w0gu_ref[...], preferred_element_type=f32)
                     + jnp.dot(mean16, w0gm_ref[...], preferred_element_type=f32)
                     + b0g_ref[...])
    g1 = jax.nn.gelu(jnp.dot(g0, w1g_ref[...], preferred_element_type=f32) + b1g_ref[...])
    unew_ref[...] = _ln(g1 + g0, sg_ref[...], bg_ref[...])


def _post_call(sums, cnts, pu2, b16, u, w0n2a, w1n2, b1n2, sn2, bn2,
               w0gu, w0gm, b0g, w1g, b1g, sg, bg):
    full = lambda shape: pl.BlockSpec(shape, lambda: (0,) * len(shape))
    return pl.pallas_call(
        _post_body,
        in_specs=[
            full((NP, H)), full((NP, H)), full((N, H)), full((N, NG)),
            full((NG, DU)),
            full((H, H)), full((H, H)), full((1, H)), full((1, H)), full((1, H)),
            full((DU, H)), full((H, H)), full((1, H)), full((H, H)), full((1, H)),
            full((1, H)), full((1, H)),
        ],
        out_specs=[full((N, H)), full((NG, H))],
        out_shape=[
            jax.ShapeDtypeStruct((N, H), f32),
            jax.ShapeDtypeStruct((NG, H), f32),
        ],
    )(sums, cnts, pu2, b16, u, w0n2a, w1n2, b1n2, sn2, bn2,
      w0gu, w0gm, b0g, w1g, b1g, sg, bg)


def kernel(x, edge_index, edge_attr, u, batch, params):
    pe, pn1, pn2, pg = params["edge"], params["node1"], params["node2"], params["glob"]
    row = edge_index[0]
    col = edge_index[1]
    b16 = jnp.broadcast_to(batch[:, None], (N, NG))

    zrow = jnp.zeros((ZR, H), f32)
    ones_rows = jnp.ones((CH, H), f32)
    cnts = _sc_counts(row, zrow, ones_rows)

    r1 = lambda v: v.reshape(1, H)
    w0e = pe["W0"]
    pre_row, pre_col, pre_u2 = _pre_call(
        x, b16, u,
        w0e[0:D], w0e[D:2 * D], w0e[2 * D + DE:], r1(pe["b0"]),
        pn1["W0"][0:D], r1(pn1["b0"]),
        pn2["W0"][H:], r1(pn2["b0"]),
    )

    gr, gc = _sc_gather(pre_row, pre_col, row, col)

    e_new, n_h = _edge_call(
        gr, gc, edge_attr,
        w0e[2 * D:2 * D + DE], pe["W1"], r1(pe["b1"]), r1(pe["ln_s"]), r1(pe["ln_b"]),
        pn1["W0"][D:], pn1["W1"], r1(pn1["b1"]), r1(pn1["ln_s"]), r1(pn1["ln_b"]),
    )

    sums = _sc_sums(n_h, row, zrow)

    x_new, u_new = _post_call(
        sums, cnts, pre_u2, b16, u,
        pn2["W0"][0:H], pn2["W1"], r1(pn2["b1"]), r1(pn2["ln_s"]), r1(pn2["ln_b"]),
        pg["W0"][0:DU], pg["W0"][DU:], r1(pg["b0"]), pg["W1"], r1(pg["b1"]),
        r1(pg["ln_s"]), r1(pg["ln_b"]),
    )
    return x_new, e_new, u_new


# speedup vs baseline: 1.1573x; 1.0320x over previous
import functools

import jax
import jax.numpy as jnp
from jax import lax
from jax.experimental import pallas as pl
from jax.experimental.pallas import tpu as pltpu
from jax.experimental.pallas import tpu_sc as plsc

N = 10000
E = 320000
D = 128
DE = 16
DU = 128
H = 128
NG = 16

NW = 32
PERW = E // NW
CH = 80
NP = 10240
NPT = NP // 16
EB = 4000
NB = 2000

f32 = jnp.float32
bf16 = jnp.bfloat16


def _pack_pair(a):
    u = lax.bitcast_convert_type(a.astype(bf16), jnp.uint16).astype(jnp.uint32)
    w = a.shape[1]
    return u[:, : w // 2] | (u[:, w // 2:] << 16)


def _unpack_lo(p):
    return lax.bitcast_convert_type(p << 16, f32)


def _unpack_hi(p):
    return lax.bitcast_convert_type(p & jnp.uint32(0xFFFF0000), f32)


def _ln(h, s, b):
    m = jnp.mean(h, axis=-1, keepdims=True)
    v = jnp.mean((h - m) ** 2, axis=-1, keepdims=True)
    return (h - m) * lax.rsqrt(v + 1e-5) * s + b


def _pre_body(x_ref, b16_ref, u_ref, wsrc_ref, wdst_ref, wu_ref, b0e_ref,
              wn1d_ref, b0n1_ref, wn2u_ref, b0n2_ref,
              prow_ref, pcol_ref, pu2_ref):
    oh = (b16_ref[...] == lax.broadcasted_iota(jnp.int32, (NB, NG), 1)).astype(f32)
    u = u_ref[...]
    x = x_ref[...]
    uproj_e = jnp.dot(u, wu_ref[...], preferred_element_type=f32)
    prow = (jnp.dot(x, wsrc_ref[...], preferred_element_type=f32)
            + jnp.dot(oh, uproj_e, preferred_element_type=f32)
            + b0e_ref[...])
    col_a = jnp.dot(x, wdst_ref[...], preferred_element_type=f32)
    col_b = jnp.dot(x, wn1d_ref[...], preferred_element_type=f32) + b0n1_ref[...]
    prow_ref[...] = prow
    pcol_ref[...] = _pack_pair(jnp.concatenate([col_a, col_b], axis=1))
    pu2_ref[...] = (jnp.dot(oh, jnp.dot(u, wn2u_ref[...], preferred_element_type=f32),
                            preferred_element_type=f32) + b0n2_ref[...])


def _pre_call(x, b16, u, wsrc, wdst, wu, b0e, wn1d, b0n1, wn2u, b0n2):
    full = lambda shape: pl.BlockSpec(shape, lambda i: (0,) * len(shape))
    return pl.pallas_call(
        _pre_body,
        grid=(N // NB,),
        in_specs=[
            pl.BlockSpec((NB, D), lambda i: (i, 0)),
            pl.BlockSpec((NB, NG), lambda i: (i, 0)),
            full((NG, DU)), full((D, H)), full((D, H)), full((DU, H)),
            full((1, H)), full((D, H)), full((1, H)), full((DU, H)), full((1, H)),
        ],
        out_specs=[
            pl.BlockSpec((NB, H), lambda i: (i, 0)),
            pl.BlockSpec((NB, H), lambda i: (i, 0)),
            pl.BlockSpec((NB, H), lambda i: (i, 0)),
        ],
        out_shape=[
            jax.ShapeDtypeStruct((N, H), f32),
            jax.ShapeDtypeStruct((N, H), jnp.uint32),
            jax.ShapeDtypeStruct((N, H), f32),
        ],
    )(x, b16, u, wsrc, wdst, wu, b0e, wn1d, b0n1, wn2u, b0n2)


GCH = 80
NCH = PERW // GCH
NPAIR = (NCH - 3) // 2


def _sc_gather(pre_row, pre_col, row, col):
    mesh = plsc.VectorSubcoreMesh(core_axis_name="c", subcore_axis_name="s")

    @functools.partial(
        pl.kernel,
        out_type=(jax.ShapeDtypeStruct((E, H), f32),
                  jax.ShapeDtypeStruct((E, H), jnp.uint32)),
        mesh=mesh,
        scratch_types=[
            pltpu.VMEM((PERW,), jnp.int32),
            pltpu.VMEM((PERW,), jnp.int32),
            pltpu.VMEM((GCH, H), f32),
            pltpu.VMEM((GCH, H), f32),
            pltpu.VMEM((GCH, H), jnp.uint32),
            pltpu.VMEM((GCH, H), jnp.uint32),
        ] + [pltpu.SemaphoreType.DMA] * 8,
    )
    def gk(prer_hbm, prec_hbm, row_hbm, col_hbm, outr_hbm, outc_hbm,
           ridx, cidx, rb0, rb1, cb0, cb1, sr0, sr1, sc0, sc1, wr0, wr1, wc0, wc1):
        wid = lax.axis_index("s") * 2 + lax.axis_index("c")
        base = wid * PERW
        pltpu.sync_copy(row_hbm.at[pl.ds(base, PERW)], ridx)
        pltpu.sync_copy(col_hbm.at[pl.ds(base, PERW)], cidx)
        rbufs, cbufs = (rb0, rb1), (cb0, cb1)
        srs, scs = (sr0, sr1), (sc0, sc1)
        wrs, wcs = (wr0, wr1), (wc0, wc1)

        def issue_g(j, s):
            pltpu.async_copy(prer_hbm.at[ridx.at[pl.ds(j * GCH, GCH)]], rbufs[s], srs[s])
            pltpu.async_copy(prec_hbm.at[cidx.at[pl.ds(j * GCH, GCH)]], cbufs[s], scs[s])

        def wait_g(s):
            pltpu.make_async_copy(prer_hbm.at[ridx.at[pl.ds(0, GCH)]], rbufs[s], srs[s]).wait()
            pltpu.make_async_copy(prec_hbm.at[cidx.at[pl.ds(0, GCH)]], cbufs[s], scs[s]).wait()

        def start_wb(j, s):
            pltpu.async_copy(rbufs[s], outr_hbm.at[pl.ds(base + j * GCH, GCH)], wrs[s])
            pltpu.async_copy(cbufs[s], outc_hbm.at[pl.ds(base + j * GCH, GCH)], wcs[s])

        def wait_wb(s):
            pltpu.make_async_copy(rbufs[s], outr_hbm.at[pl.ds(base, GCH)], wrs[s]).wait()
            pltpu.make_async_copy(cbufs[s], outc_hbm.at[pl.ds(base, GCH)], wcs[s]).wait()

        issue_g(0, 0)
        issue_g(1, 1)

        def body(jj, carry):
            j0 = jj * 2
            wait_g(0)
            start_wb(j0, 0)
            wait_g(1)
            start_wb(j0 + 1, 1)
            wait_wb(0)
            issue_g(j0 + 2, 0)
            wait_wb(1)
            issue_g(j0 + 3, 1)
            return carry

        lax.fori_loop(0, NPAIR, body, 0)
        wait_g(0)
        start_wb(NCH - 3, 0)
        wait_g(1)
        start_wb(NCH - 2, 1)
        wait_wb(0)
        issue_g(NCH - 1, 0)
        wait_wb(1)
        wait_g(0)
        start_wb(NCH - 1, 0)
        wait_wb(0)

    return gk(pre_row, pre_col, row, col)


def _edge_body(gr_ref, gc_ref, ea_ref, w0ea_ref, w1e_ref, b1e_ref, se_ref, be_ref,
               w0n1e_ref, w1n1_ref, b1n1_ref, sn1_ref, bn1_ref,
               enew_ref, nh_ref):
    gcp = gc_ref[...]
    gca = _unpack_lo(gcp)
    gcb = _unpack_hi(gcp)
    eap = jnp.dot(ea_ref[...], w0ea_ref[...], preferred_element_type=f32)
    h0 = jax.nn.gelu(gr_ref[...] + gca + eap)
    h1 = jax.nn.gelu(jnp.dot(h0, w1e_ref[...], preferred_element_type=f32) + b1e_ref[...])
    e_new = _ln(h1 + h0, se_ref[...], be_ref[...])
    m0 = jax.nn.gelu(gcb + jnp.dot(e_new, w0n1e_ref[...], preferred_element_type=f32))
    m1 = jax.nn.gelu(jnp.dot(m0, w1n1_ref[...], preferred_element_type=f32) + b1n1_ref[...])
    enew_ref[...] = e_new
    nh_ref[...] = _ln(m1 + m0, sn1_ref[...], bn1_ref[...])


def _edge_call(gr, gc, edge_attr, w0ea, w1e, b1e, se, be, w0n1e, w1n1, b1n1, sn1, bn1):
    full = lambda shape: pl.BlockSpec(shape, lambda i: (0,) * len(shape))
    return pl.pallas_call(
        _edge_body,
        grid=(E // EB,),
        in_specs=[
            pl.BlockSpec((EB, H), lambda i: (i, 0)),
            pl.BlockSpec((EB, H), lambda i: (i, 0)),
            pl.BlockSpec((EB, DE), lambda i: (i, 0)),
            full((DE, H)), full((H, H)), full((1, H)), full((1, H)), full((1, H)),
            full((H, H)), full((H, H)), full((1, H)), full((1, H)), full((1, H)),
        ],
        out_specs=[
            pl.BlockSpec((EB, H), lambda i: (i, 0)),
            pl.BlockSpec((EB, H), lambda i: (i, 0)),
        ],
        out_shape=[
            jax.ShapeDtypeStruct((E, H), f32),
            jax.ShapeDtypeStruct((E, H), f32),
        ],
    )(gr, gc, edge_attr, w0ea, w1e, b1e, se, be, w0n1e, w1n1, b1n1, sn1, bn1)


ZR = 128
HALF = NP // 2
ACC = 6144
TRASH = HALF
PERT = E // 16


NCHS = PERT // CH


def _scatter_common(cid, sid, idxall, idxb2, vbuf, zrb, acc_sh):
    lo = cid * HALF

    def zero_acc():
        def zbody(k, carry):
            pltpu.sync_copy(zrb, acc_sh.at[pl.ds(sid * (ACC // 16) + k * ZR, ZR)])
            return carry

        lax.fori_loop(0, ACC // 16 // ZR, zbody, 0)

    def remap(j):
        for k in range(CH // 16):
            v = idxall[pl.ds(j * CH + k * 16, 16)] - lo
            oob = (v < 0) | (v >= HALF)
            idxb2[pl.ds(k * 16, 16)] = jnp.where(oob, TRASH, v)

    def drain(dst_hbm):
        def dbody(k, carry):
            src = sid * (HALF // 16) + k * CH
            pltpu.sync_copy(acc_sh.at[pl.ds(src, CH)], vbuf)
            pltpu.sync_copy(vbuf, dst_hbm.at[pl.ds(cid * HALF + src, CH)])
            return carry

        lax.fori_loop(0, HALF // 16 // CH, dbody, 0)

    return zero_acc, remap, drain


def _sc_sums(nh, row, zrow):
    mesh = plsc.VectorSubcoreMesh(core_axis_name="c", subcore_axis_name="s")

    @functools.partial(
        pl.kernel,
        out_type=jax.ShapeDtypeStruct((NP, H), f32),
        mesh=mesh,
        scratch_types=[
            pltpu.VMEM((PERT,), jnp.int32),
            pltpu.VMEM((CH,), jnp.int32),
            pltpu.VMEM((CH, H), f32),
            pltpu.VMEM((CH, H), f32),
            pltpu.VMEM((ZR, H), f32),
            pltpu.VMEM_SHARED((ACC, H), f32),
            pltpu.SemaphoreType.DMA,
            pltpu.SemaphoreType.DMA,
        ],
    )
    def sk(nh_hbm, row_hbm, zrow_hbm, outs_hbm,
           idxall, idxb2, vb0, vb1, zrb, acc_sh, sl0, sl1):
        cid = lax.axis_index("c")
        sid = lax.axis_index("s")
        base = sid * PERT
        pltpu.sync_copy(zrow_hbm, zrb)
        pltpu.sync_copy(row_hbm.at[pl.ds(base, PERT)], idxall)
        zero_acc, remap, drain = _scatter_common(cid, sid, idxall, idxb2, vb0, zrb, acc_sh)
        zero_acc()
        plsc.subcore_barrier()
        vbufs, sls = (vb0, vb1), (sl0, sl1)

        def load(j, s):
            pltpu.async_copy(nh_hbm.at[pl.ds(base + j * CH, CH)], vbufs[s], sls[s])

        def wait_load(s):
            pltpu.make_async_copy(nh_hbm.at[pl.ds(base, CH)], vbufs[s], sls[s]).wait()

        def scat(s):
            pltpu.sync_copy(vbufs[s], acc_sh.at[idxb2], add=True)

        load(0, 0)

        def body(jj, carry):
            j0 = jj * 2
            wait_load(0)
            load(j0 + 1, 1)
            remap(j0)
            scat(0)
            wait_load(1)
            load(j0 + 2, 0)
            remap(j0 + 1)
            scat(1)
            return carry

        lax.fori_loop(0, NCHS // 2 - 1, body, 0)
        wait_load(0)
        load(NCHS - 1, 1)
        remap(NCHS - 2)
        scat(0)
        wait_load(1)
        remap(NCHS - 1)
        scat(1)
        plsc.subcore_barrier()
        drain(outs_hbm)

    return sk(nh, row, zrow)


def _sc_counts(row, zrow, ones_rows):
    mesh = plsc.VectorSubcoreMesh(core_axis_name="c", subcore_axis_name="s")

    @functools.partial(
        pl.kernel,
        out_type=jax.ShapeDtypeStruct((NP, H), f32),
        mesh=mesh,
        scratch_types=[
            pltpu.VMEM((PERT,), jnp.int32),
            pltpu.VMEM((CH,), jnp.int32),
            pltpu.VMEM((CH, H), f32),
            pltpu.VMEM((ZR, H), f32),
            pltpu.VMEM_SHARED((ACC, H), f32),
            pltpu.VMEM((CH, H), f32),
        ],
    )
    def ck(row_hbm, zrow_hbm, ones_hbm, outc_hbm,
           idxall, idxb2, vbuf, zrb, acc_sh, onesb):
        cid = lax.axis_index("c")
        sid = lax.axis_index("s")
        pltpu.sync_copy(zrow_hbm, zrb)
        pltpu.sync_copy(ones_hbm, onesb)
        pltpu.sync_copy(row_hbm.at[pl.ds(sid * PERT, PERT)], idxall)
        zero_acc, remap, drain = _scatter_common(cid, sid, idxall, idxb2, vbuf, zrb, acc_sh)
        zero_acc()
        plsc.subcore_barrier()

        def body(j, carry):
            remap(j)
            pltpu.sync_copy(onesb, acc_sh.at[idxb2], add=True)
            return carry

        lax.fori_loop(0, NCHS, body, 0)
        plsc.subcore_barrier()
        drain(outc_hbm)

    return ck(row, zrow, ones_rows)


def _post_body(sums_ref, cnts_ref, pu2_ref, b16_ref, u_ref,
               w0n2a_ref, w1n2_ref, b1n2_ref, sn2_ref, bn2_ref,
               w0gu_ref, w0gm_ref, b0g_ref, w1g_ref, b1g_ref, sg_ref, bg_ref,
               xnew_ref, unew_ref):
    s = sums_ref[:N, :]
    c = cnts_ref[:N, 0:1]
    agg = s * (1.0 / jnp.maximum(c, 1.0))
    a0 = jax.nn.gelu(jnp.dot(agg, w0n2a_ref[...], preferred_element_type=f32) + pu2_ref[...])
    a1 = jax.nn.gelu(jnp.dot(a0, w1n2_ref[...], preferred_element_type=f32) + b1n2_ref[...])
    x_new = _ln(a1 + a0, sn2_ref[...], bn2_ref[...])
    xnew_ref[...] = x_new
    oh = (b16_ref[...] == lax.broadcasted_iota(jnp.int32, (N, NG), 1)).astype(f32)
    cnt16 = jnp.sum(oh, axis=0, keepdims=True)
    ohn = oh * (1.0 / jnp.maximum(cnt16, 1.0))
    mean16 = lax.dot_general(ohn, x_new, (((0,), (0,)), ((), ())),
                             preferred_element_type=f32)
    u = u_ref[...]
    g0 = jax.nn.gelu(jnp.dot(u, w0gu_ref[...], preferred_element_type=f32)
                     + jnp.dot(mean16, w0gm_ref[...], preferred_element_type=f32)
                     + b0g_ref[...])
    g1 = jax.nn.gelu(jnp.dot(g0, w1g_ref[...], preferred_element_type=f32) + b1g_ref[...])
    unew_ref[...] = _ln(g1 + g0, sg_ref[...], bg_ref[...])


def _post_call(sums, cnts, pu2, b16, u, w0n2a, w1n2, b1n2, sn2, bn2,
               w0gu, w0gm, b0g, w1g, b1g, sg, bg):
    full = lambda shape: pl.BlockSpec(shape, lambda: (0,) * len(shape))
    return pl.pallas_call(
        _post_body,
        in_specs=[
            full((NP, H)), full((NP, H)), full((N, H)), full((N, NG)),
            full((NG, DU)),
            full((H, H)), full((H, H)), full((1, H)), full((1, H)), full((1, H)),
            full((DU, H)), full((H, H)), full((1, H)), full((H, H)), full((1, H)),
            full((1, H)), full((1, H)),
        ],
        out_specs=[full((N, H)), full((NG, H))],
        out_shape=[
            jax.ShapeDtypeStruct((N, H), f32),
            jax.ShapeDtypeStruct((NG, H), f32),
        ],
    )(sums, cnts, pu2, b16, u, w0n2a, w1n2, b1n2, sn2, bn2,
      w0gu, w0gm, b0g, w1g, b1g, sg, bg)


def kernel(x, edge_index, edge_attr, u, batch, params):
    pe, pn1, pn2, pg = params["edge"], params["node1"], params["node2"], params["glob"]
    row = edge_index[0]
    col = edge_index[1]
    b16 = jnp.broadcast_to(batch[:, None], (N, NG))

    zrow = jnp.zeros((ZR, H), f32)
    ones_rows = jnp.ones((CH, H), f32)
    cnts = _sc_counts(row, zrow, ones_rows)

    r1 = lambda v: v.reshape(1, H)
    w0e = pe["W0"]
    pre_row, pre_col, pre_u2 = _pre_call(
        x, b16, u,
        w0e[0:D], w0e[D:2 * D], w0e[2 * D + DE:], r1(pe["b0"]),
        pn1["W0"][0:D], r1(pn1["b0"]),
        pn2["W0"][H:], r1(pn2["b0"]),
    )

    gr, gc = _sc_gather(pre_row, pre_col, row, col)

    e_new, n_h = _edge_call(
        gr, gc, edge_attr,
        w0e[2 * D:2 * D + DE], pe["W1"], r1(pe["b1"]), r1(pe["ln_s"]), r1(pe["ln_b"]),
        pn1["W0"][D:], pn1["W1"], r1(pn1["b1"]), r1(pn1["ln_s"]), r1(pn1["ln_b"]),
    )

    sums = _sc_sums(n_h, row, zrow)

    x_new, u_new = _post_call(
        sums, cnts, pre_u2, b16, u,
        pn2["W0"][0:H], pn2["W1"], r1(pn2["b1"]), r1(pn2["ln_s"]), r1(pn2["ln_b"]),
        pg["W0"][0:DU], pg["W0"][DU:], r1(pg["b0"]), pg["W1"], r1(pg["b1"]),
        r1(pg["ln_s"]), r1(pg["ln_b"]),
    )
    return x_new, e_new, u_new
